# baseline (speedup 1.0000x reference)
import functools

import jax
import jax.numpy as jnp
from jax import lax
from jax.experimental import pallas as pl
from jax.experimental.pallas import tpu as pltpu
from jax.experimental.pallas import tpu_sc as plsc

B, S, H = 1024, 200, 128
N = B * S
NUM_WORKERS = 32
ROWS_PER_W = N // NUM_WORKERS
CHUNK = 80
N_CHUNKS = ROWS_PER_W // CHUNK
NBUF = 8

_mesh = plsc.VectorSubcoreMesh(core_axis_name="c", subcore_axis_name="s")


@functools.partial(
    pl.kernel,
    mesh=_mesh,
    out_type=jax.ShapeDtypeStruct((N, H), jnp.float32),
    scratch_types=(
        [pltpu.VMEM((N_CHUNKS, CHUNK), jnp.int32)]
        + [pltpu.VMEM((CHUNK, H), jnp.float32) for _ in range(NBUF)]
        + [pltpu.SemaphoreType.DMA for _ in range(2 * NBUF)]
    ),
)
def _gather_kernel(idx_hbm, table_hbm, out_hbm, idx_v, *rest):
    bufs = rest[:NBUF]
    gs = rest[NBUF:2 * NBUF]
    ws = rest[2 * NBUF:]
    wid = lax.axis_index("s") * 2 + lax.axis_index("c")
    base = wid * ROWS_PER_W
    pltpu.sync_copy(idx_hbm.at[wid], idx_v)

    def gather_desc(c, buf, sem):
        return pltpu.make_async_copy(table_hbm.at[idx_v.at[c]], buf, sem)

    def write_desc(c, buf, sem):
        return pltpu.make_async_copy(
            buf, out_hbm.at[pl.ds(base + c * CHUNK, CHUNK)], sem)

    for b in range(NBUF):
        gather_desc(b, bufs[b], gs[b]).start()

    def body(i, _):
        cbase = i * NBUF
        for b in range(NBUF):
            c = cbase + b
            gather_desc(c, bufs[b], gs[b]).wait()
            write_desc(c, bufs[b], ws[b]).start()
        for b in range(NBUF):
            c = cbase + b + NBUF

            @pl.when(c < N_CHUNKS)
            def _(c=c, b=b):
                write_desc(c - NBUF, bufs[b], ws[b]).wait()
                gather_desc(c, bufs[b], gs[b]).start()

        return ()

    lax.fori_loop(0, N_CHUNKS // NBUF, body, (), unroll=False)

    cL = N_CHUNKS - NBUF
    for b in range(NBUF):
        write_desc(cL + b, bufs[b], ws[b]).wait()


def kernel(x, embedding):
    idx = x.reshape(NUM_WORKERS, N_CHUNKS, CHUNK)
    out = _gather_kernel(idx, embedding)
    return out.reshape(B, S, H)


# speedup vs baseline: 1.0147x; 1.0147x over previous
import functools

import jax
import jax.numpy as jnp
from jax import lax
from jax.experimental import pallas as pl
from jax.experimental.pallas import tpu as pltpu
from jax.experimental.pallas import tpu_sc as plsc

B, S, H = 1024, 200, 128
N = B * S
NUM_WORKERS = 32
ROWS_PER_W = N // NUM_WORKERS
CHUNK = 64
N_CHUNKS = ROWS_PER_W // CHUNK
NBUF = 10
NPAIR = NBUF // 2

_mesh = plsc.VectorSubcoreMesh(core_axis_name="c", subcore_axis_name="s")


@functools.partial(
    pl.kernel,
    mesh=_mesh,
    out_type=jax.ShapeDtypeStruct((N, H), jnp.float32),
    scratch_types=(
        [pltpu.VMEM((N_CHUNKS, CHUNK), jnp.int32)]
        + [pltpu.VMEM((NBUF * CHUNK, H), jnp.float32)]
        + [pltpu.SemaphoreType.DMA for _ in range(NBUF + NPAIR)]
    ),
)
def _gather_kernel(idx_hbm, table_hbm, out_hbm, idx_v, buf, *sems):
    gs = sems[:NBUF]
    ws = sems[NBUF:]
    wid = lax.axis_index("s") * 2 + lax.axis_index("c")
    base = wid * ROWS_PER_W
    pltpu.sync_copy(idx_hbm.at[wid], idx_v)

    def gather_desc(c, b):
        return pltpu.make_async_copy(
            table_hbm.at[idx_v.at[c]], buf.at[pl.ds(b * CHUNK, CHUNK)], gs[b])

    def write_desc(c, p):
        return pltpu.make_async_copy(
            buf.at[pl.ds(2 * p * CHUNK, 2 * CHUNK)],
            out_hbm.at[pl.ds(base + c * CHUNK, 2 * CHUNK)],
            ws[p])

    for b in range(NBUF):
        gather_desc(b, b).start()

    def body(i, _):
        cbase = i * NBUF
        for p in range(NPAIR):
            b = 2 * p
            gather_desc(cbase + b, b).wait()
            gather_desc(cbase + b + 1, b + 1).wait()
            write_desc(cbase + b, p).start()
        for p in range(NPAIR):
            b = 2 * p
            c = cbase + b + NBUF

            @pl.when(c < N_CHUNKS)
            def _(c=c, b=b, p=p):
                write_desc(c - NBUF, p).wait()
                gather_desc(c, b).start()
                gather_desc(c + 1, b + 1).start()

        return ()

    lax.fori_loop(0, N_CHUNKS // NBUF, body, (), unroll=False)

    cL = N_CHUNKS - NBUF
    for p in range(NPAIR):
        write_desc(cL + 2 * p, p).wait()


def kernel(x, embedding):
    idx = x.reshape(NUM_WORKERS, N_CHUNKS, CHUNK)
    out = _gather_kernel(idx, embedding)
    return out.reshape(B, S, H)


# speedup vs baseline: 1.0168x; 1.0021x over previous
import functools

import jax
import jax.numpy as jnp
from jax import lax
from jax.experimental import pallas as pl
from jax.experimental.pallas import tpu as pltpu
from jax.experimental.pallas import tpu_sc as plsc

B, S, H = 1024, 200, 128
N = B * S
NUM_WORKERS = 32
ROWS_PER_W = N // NUM_WORKERS
CHUNK = 64
N_CHUNKS = ROWS_PER_W // CHUNK
NBUF = 10
NPAIR = NBUF // 5

_mesh = plsc.VectorSubcoreMesh(core_axis_name="c", subcore_axis_name="s")


@functools.partial(
    pl.kernel,
    mesh=_mesh,
    out_type=jax.ShapeDtypeStruct((N, H), jnp.float32),
    scratch_types=(
        [pltpu.VMEM((N_CHUNKS, CHUNK), jnp.int32)]
        + [pltpu.VMEM((NBUF * CHUNK, H), jnp.float32)]
        + [pltpu.SemaphoreType.DMA for _ in range(NBUF + NPAIR)]
    ),
)
def _gather_kernel(idx_hbm, table_hbm, out_hbm, idx_v, buf, *sems):
    gs = sems[:NBUF]
    ws = sems[NBUF:]
    wid = lax.axis_index("s") * 2 + lax.axis_index("c")
    base = wid * ROWS_PER_W
    pltpu.sync_copy(idx_hbm.at[wid], idx_v)

    def gather_desc(c, b):
        return pltpu.make_async_copy(
            table_hbm.at[idx_v.at[c]], buf.at[pl.ds(b * CHUNK, CHUNK)], gs[b])

    def write_desc(c, p):
        return pltpu.make_async_copy(
            buf.at[pl.ds(5 * p * CHUNK, 5 * CHUNK)],
            out_hbm.at[pl.ds(base + c * CHUNK, 5 * CHUNK)],
            ws[p])

    for b in range(NBUF):
        gather_desc(b, b).start()

    def body(i, _):
        cbase = i * NBUF
        for p in range(NPAIR):
            b = 5 * p
            for j in range(5):
                gather_desc(cbase + b + j, b + j).wait()
            write_desc(cbase + b, p).start()
        for p in range(NPAIR):
            b = 5 * p
            c = cbase + b + NBUF

            @pl.when(c < N_CHUNKS)
            def _(c=c, b=b, p=p):
                write_desc(c - NBUF, p).wait()
                for j in range(5):
                    gather_desc(c + j, b + j).start()

        return ()

    lax.fori_loop(0, N_CHUNKS // NBUF, body, (), unroll=False)

    cL = N_CHUNKS - NBUF
    for p in range(NPAIR):
        write_desc(cL + 5 * p, p).wait()


def kernel(x, embedding):
    idx = x.reshape(NUM_WORKERS, N_CHUNKS, CHUNK)
    out = _gather_kernel(idx, embedding)
    return out.reshape(B, S, H)
